# baseline (speedup 1.0000x reference)
import functools

import jax
import jax.numpy as jnp
from jax import lax
from jax.experimental import pallas as pl
from jax.experimental.pallas import tpu as pltpu
from jax.experimental.pallas import tpu_sc as plsc

EMBED_DIM = 64
LANES = 16
NUM_CORES = 2
NUM_SUBCORES = 16
NW = NUM_CORES * NUM_SUBCORES
EPW = EMBED_DIM // NW

_mesh = plsc.VectorSubcoreMesh(core_axis_name="c", subcore_axis_name="s")


def _make_gather(batch: int, hist: int, vocab: int):
  nchunk = batch // LANES
  assert hist % 2 == 0

  @functools.partial(
      pl.kernel,
      mesh=_mesh,
      compiler_params=pltpu.CompilerParams(needs_layout_passes=False),
      out_type=jax.ShapeDtypeStruct((hist, EMBED_DIM, batch), jnp.float32),
      scratch_types=[
          pltpu.VMEM((vocab,), jnp.float32),
          pltpu.VMEM((2, batch), jnp.int32),
          pltpu.VMEM((2, batch), jnp.float32),
      ]
      + [pltpu.SemaphoreType.DMA] * 4,
  )
  def gather_kernel(xt_hbm, tt_hbm, out_hbm, rowbuf, idxbuf, outbuf, *sems):
    isem = sems[:2]
    osem = sems[2:]
    wid = lax.axis_index("s") * NUM_CORES + lax.axis_index("c")

    def fire_idx(h, p):
      pltpu.async_copy(xt_hbm.at[h], idxbuf.at[p], isem[p])

    def wait_idx(h, p):
      pltpu.make_async_copy(xt_hbm.at[h], idxbuf.at[p], isem[p]).wait()

    def fire_out(h, e, p):
      pltpu.async_copy(outbuf.at[p], out_hbm.at[h, e], osem[p])

    def wait_out(h, e, p):
      pltpu.make_async_copy(outbuf.at[p], out_hbm.at[h, e], osem[p]).wait()

    for ei in range(EPW):
      e = wid * EPW + ei
      pltpu.sync_copy(tt_hbm.at[e], rowbuf)
      fire_idx(0, 0)

      def body(g, carry, ei=ei, e=e):
        for p in range(2):
          h = g * 2 + p
          pl.when(h + 1 <= hist - 1)(lambda: fire_idx(h + 1, (p + 1) % 2))
          wait_idx(h, p)
          if ei == 0:
            pl.when(g >= 1)(lambda: wait_out(h - 2, e, p))
          else:
            pl.when(g >= 1)(lambda: wait_out(h - 2, e, p))
            pl.when(g == 0)(lambda: wait_out(hist - 2 + p, e - 1, p))
          G = 16
          for kg in range(nchunk // G):
            sls = [pl.ds((kg * G + i) * LANES, LANES) for i in range(G)]
            idxs = [idxbuf[p, sl] for sl in sls]
            vals = [plsc.load_gather(rowbuf, [ix]) for ix in idxs]
            for sl, v in zip(sls, vals):
              outbuf[p, sl] = v
          fire_out(h, e, p)
        return carry

      lax.fori_loop(0, hist // 2, body, 0, unroll=False)

    e_last = wid * EPW + EPW - 1
    for p in range(2):
      wait_out(hist - 2 + p, e_last, p)

  return gather_kernel


def kernel(x, table):
  batch, hist = x.shape
  vocab, _ = table.shape
  out = _make_gather(batch, hist, vocab)(x.T, table.T)
  return out.transpose(2, 0, 1)


# speedup vs baseline: 1.0056x; 1.0056x over previous
import functools

import jax
import jax.numpy as jnp
from jax import lax
from jax.experimental import pallas as pl
from jax.experimental.pallas import tpu as pltpu
from jax.experimental.pallas import tpu_sc as plsc

EMBED_DIM = 64
LANES = 16
NUM_CORES = 2
NUM_SUBCORES = 16
NW = NUM_CORES * NUM_SUBCORES
EPW = EMBED_DIM // NW

_mesh = plsc.VectorSubcoreMesh(core_axis_name="c", subcore_axis_name="s")


def _make_gather(batch: int, hist: int, vocab: int):
  nchunk = batch // LANES
  assert hist % 2 == 0

  @functools.partial(
      pl.kernel,
      mesh=_mesh,
      compiler_params=pltpu.CompilerParams(needs_layout_passes=False),
      out_type=jax.ShapeDtypeStruct((hist, EMBED_DIM, batch), jnp.float32),
      scratch_types=[
          pltpu.VMEM((vocab,), jnp.float32),
          pltpu.VMEM((2, batch), jnp.int32),
          pltpu.VMEM((2, batch), jnp.float32),
      ]
      + [pltpu.SemaphoreType.DMA] * 4,
  )
  def gather_kernel(xt_hbm, tt_hbm, out_hbm, rowbuf, idxbuf, outbuf, *sems):
    isem = sems[:2]
    osem = sems[2:]
    wid = lax.axis_index("s") * NUM_CORES + lax.axis_index("c")

    def fire_idx(h, p):
      pltpu.async_copy(xt_hbm.at[h], idxbuf.at[p], isem[p])

    def wait_idx(h, p):
      pltpu.make_async_copy(xt_hbm.at[h], idxbuf.at[p], isem[p]).wait()

    def fire_out(h, e, p):
      pltpu.async_copy(outbuf.at[p], out_hbm.at[h, e], osem[p])

    def wait_out(h, e, p):
      pltpu.make_async_copy(outbuf.at[p], out_hbm.at[h, e], osem[p]).wait()

    for ei in range(EPW):
      e = wid * EPW + ei
      pltpu.sync_copy(tt_hbm.at[e], rowbuf)
      fire_idx(0, 0)

      def body(g, carry, ei=ei, e=e):
        for p in range(2):
          h = g * 2 + p
          pl.when(h + 1 <= hist - 1)(lambda: fire_idx(h + 1, (p + 1) % 2))
          wait_idx(h, p)
          if ei == 0:
            pl.when(g >= 1)(lambda: wait_out(h - 2, e, p))
          else:
            pl.when(g >= 1)(lambda: wait_out(h - 2, e, p))
            pl.when(g == 0)(lambda: wait_out(hist - 2 + p, e - 1, p))
          G = 8
          for kg in range(nchunk // G):
            sls = [pl.ds((kg * G + i) * LANES, LANES) for i in range(G)]
            idxs = [idxbuf[p, sl] for sl in sls]
            vals = [plsc.load_gather(rowbuf, [ix]) for ix in idxs]
            for sl, v in zip(sls, vals):
              outbuf[p, sl] = v
          fire_out(h, e, p)
        return carry

      lax.fori_loop(0, hist // 2, body, 0, unroll=False)

    e_last = wid * EPW + EPW - 1
    for p in range(2):
      wait_out(hist - 2 + p, e_last, p)

  return gather_kernel


def kernel(x, table):
  batch, hist = x.shape
  vocab, _ = table.shape
  out = _make_gather(batch, hist, vocab)(x.T, table.T)
  return out.transpose(2, 0, 1)


# speedup vs baseline: 1.0392x; 1.0335x over previous
import functools

import jax
import jax.numpy as jnp
from jax import lax
from jax.experimental import pallas as pl
from jax.experimental.pallas import tpu as pltpu
from jax.experimental.pallas import tpu_sc as plsc

EMBED_DIM = 64
LANES = 16
NUM_CORES = 2
NUM_SUBCORES = 16
NW = NUM_CORES * NUM_SUBCORES
EPW = EMBED_DIM // NW

_mesh = plsc.VectorSubcoreMesh(core_axis_name="c", subcore_axis_name="s")


def _make_gather(batch: int, hist: int, vocab: int):
  nchunk = batch // LANES
  assert hist % 2 == 0

  @functools.partial(
      pl.kernel,
      mesh=_mesh,
      compiler_params=pltpu.CompilerParams(needs_layout_passes=False),
      out_type=jax.ShapeDtypeStruct((hist, EMBED_DIM, batch), jnp.float32),
      scratch_types=[
          pltpu.VMEM((vocab,), jnp.float32),
          pltpu.VMEM((2, batch), jnp.int32),
          pltpu.VMEM((2, batch), jnp.float32),
      ]
      + [pltpu.SemaphoreType.DMA] * 4,
  )
  def gather_kernel(xt_hbm, tt_hbm, out_hbm, rowbuf, idxbuf, outbuf, *sems):
    isem = sems[:2]
    osem = sems[2:]
    wid = lax.axis_index("s") * NUM_CORES + lax.axis_index("c")

    def fire_idx(h, p):
      pltpu.async_copy(xt_hbm.at[h], idxbuf.at[p], isem[p])

    def wait_idx(h, p):
      pltpu.make_async_copy(xt_hbm.at[h], idxbuf.at[p], isem[p]).wait()

    def fire_out(h, e, p):
      pltpu.async_copy(outbuf.at[p], out_hbm.at[h, e], osem[p])

    def wait_out(h, e, p):
      pltpu.make_async_copy(outbuf.at[p], out_hbm.at[h, e], osem[p]).wait()

    for ei in range(EPW):
      e = wid * EPW + ei
      pltpu.sync_copy(tt_hbm.at[e], rowbuf)
      fire_idx(0, 0)

      def body(g, carry, ei=ei, e=e):
        for p in range(2):
          h = g * 2 + p
          pl.when(h + 1 <= hist - 1)(lambda: fire_idx(h + 1, (p + 1) % 2))
          wait_idx(h, p)
          if ei == 0:
            pl.when(g >= 1)(lambda: wait_out(h - 2, e, p))
          else:
            pl.when(g >= 1)(lambda: wait_out(h - 2, e, p))
            pl.when(g == 0)(lambda: wait_out(hist - 2 + p, e - 1, p))
          @plsc.parallel_loop(0, nchunk, step=1, unroll=8)
          def _(k, p=p):
            sl = pl.ds(k * LANES, LANES)
            outbuf[p, sl] = plsc.load_gather(rowbuf, [idxbuf[p, sl]])
          fire_out(h, e, p)
        return carry

      lax.fori_loop(0, hist // 2, body, 0, unroll=False)

    e_last = wid * EPW + EPW - 1
    for p in range(2):
      wait_out(hist - 2 + p, e_last, p)

  return gather_kernel


def kernel(x, table):
  batch, hist = x.shape
  vocab, _ = table.shape
  out = _make_gather(batch, hist, vocab)(x.T, table.T)
  return out.transpose(2, 0, 1)
